# baseline (speedup 1.0000x reference)
import functools

import jax
import jax.numpy as jnp
from jax import lax
from jax.experimental import pallas as pl
from jax.experimental.pallas import tpu as pltpu
from jax.experimental.pallas import tpu_sc as plsc

VOCAB = 1000000
D_MODEL = 64
N_IDX = 4096 * 200

_NC, _NS = 2, 16
_NW = _NC * _NS
_B_PER_W = N_IDX // _NW
_CHUNK = 128
_N_CHUNKS = _B_PER_W // _CHUNK

_mesh = plsc.VectorSubcoreMesh(core_axis_name="c", subcore_axis_name="s")


@functools.partial(
    pl.kernel,
    out_type=jax.ShapeDtypeStruct((N_IDX, D_MODEL), jnp.float32),
    mesh=_mesh,
    scratch_types=[
        pltpu.VMEM((_B_PER_W,), jnp.int32),
        pltpu.VMEM((_CHUNK, D_MODEL), jnp.float32),
        pltpu.SemaphoreType.DMA,
    ],
    compiler_params=pltpu.CompilerParams(use_tc_tiling_on_sc=False),
)
def _embed(table_hbm, idx_hbm, out_hbm, idx_v, rows_v, sem):
    wid = lax.axis_index("s") * _NC + lax.axis_index("c")
    base = wid * _B_PER_W
    pltpu.sync_copy(idx_hbm.at[pl.ds(base, _B_PER_W)], idx_v)

    def body(g, carry):
        off = g * _CHUNK
        pltpu.async_copy(
            table_hbm.at[idx_v.at[pl.ds(off, _CHUNK)]], rows_v, sem
        ).wait()
        pltpu.sync_copy(rows_v, out_hbm.at[pl.ds(base + off, _CHUNK)])
        return carry

    lax.fori_loop(0, _N_CHUNKS, body, 0)


def kernel(x, table):
    xf = x.reshape(-1).astype(jnp.int32)
    out = _embed(table, xf)
    return out.reshape(x.shape + (D_MODEL,))


# speedup vs baseline: 1.1165x; 1.1165x over previous
import functools

import jax
import jax.numpy as jnp
from jax import lax
from jax.experimental import pallas as pl
from jax.experimental.pallas import tpu as pltpu
from jax.experimental.pallas import tpu_sc as plsc

VOCAB = 1000000
D_MODEL = 64
N_IDX = 4096 * 200

_NC, _NS = 2, 16
_NW = _NC * _NS
_B_PER_W = N_IDX // _NW
_BLK = 512
_NBLK = _B_PER_W // _BLK

_mesh = plsc.VectorSubcoreMesh(core_axis_name="c", subcore_axis_name="s")


@functools.partial(
    pl.kernel,
    out_type=jax.ShapeDtypeStruct((N_IDX, D_MODEL), jnp.float32),
    mesh=_mesh,
    scratch_types=[
        pltpu.VMEM((_B_PER_W,), jnp.int32),
        pltpu.VMEM((2, _BLK, D_MODEL), jnp.float32),
        pltpu.SemaphoreType.DMA((2,)),
        pltpu.SemaphoreType.DMA((2,)),
    ],
    compiler_params=pltpu.CompilerParams(use_tc_tiling_on_sc=False),
)
def _embed(table_hbm, idx_hbm, out_hbm, idx_v, rows_v, gsem, osem):
    wid = lax.axis_index("s") * _NC + lax.axis_index("c")
    base = wid * _B_PER_W
    pltpu.sync_copy(idx_hbm.at[pl.ds(base, _B_PER_W)], idx_v)

    def gather_desc(g, b):
        return pltpu.make_async_copy(
            table_hbm.at[idx_v.at[pl.ds(g * _BLK, _BLK)]],
            rows_v.at[b],
            gsem.at[b],
        )

    def out_desc(g, b):
        return pltpu.make_async_copy(
            rows_v.at[b],
            out_hbm.at[pl.ds(base + g * _BLK, _BLK)],
            osem.at[b],
        )

    gather_desc(0, 0).start()

    def body(g, carry):
        b = lax.rem(g, 2)
        nb = 1 - b

        @pl.when(g + 1 < _NBLK)
        def _fire_next():
            @pl.when(g >= 1)
            def _drain_prev_out():
                out_desc(g - 1, nb).wait()

            gather_desc(g + 1, nb).start()

        gather_desc(g, b).wait()
        out_desc(g, b).start()
        return carry

    lax.fori_loop(0, _NBLK, body, 0)
    out_desc(_NBLK - 2, (_NBLK - 2) % 2).wait()
    out_desc(_NBLK - 1, (_NBLK - 1) % 2).wait()


def kernel(x, table):
    xf = x.reshape(-1).astype(jnp.int32)
    out = _embed(table, xf)
    return out.reshape(x.shape + (D_MODEL,))
